# baseline (speedup 1.0000x reference)
import functools

import jax
import jax.numpy as jnp
from jax import lax
from jax.experimental import pallas as pl
from jax.experimental.pallas import tpu as pltpu
from jax.experimental.pallas import tpu_sc as plsc


def _gate_body(x_ref, wg_ref, g1_ref, g2_ref, i1_ref, i2_ref, l1_ref, l2_ref,
               loss_ref, ce_ref, cnt1_ref, cnt2_ref, me_ref):
    step = pl.program_id(0)
    nb = pl.num_programs(0)
    bt = x_ref.shape[0]
    ne = wg_ref.shape[0]

    @pl.when(step == 0)
    def _init():
        cnt1_ref[...] = jnp.zeros_like(cnt1_ref)
        cnt2_ref[...] = jnp.zeros_like(cnt2_ref)
        me_ref[...] = jnp.zeros_like(me_ref)

    logits = lax.dot_general(
        x_ref[...].astype(jnp.bfloat16), wg_ref[...].astype(jnp.bfloat16),
        (((1,), (1,)), ((), ())), preferred_element_type=jnp.float32)

    col = lax.broadcasted_iota(jnp.int32, (bt, ne), 1)
    m1 = jnp.max(logits, axis=1, keepdims=True)
    i1 = jnp.min(jnp.where(logits == m1, col, ne), axis=1)
    onehot1 = col == i1[:, None]
    masked = jnp.where(onehot1, -jnp.inf, logits)
    m2 = jnp.max(masked, axis=1, keepdims=True)
    i2 = jnp.min(jnp.where(masked == m2, col, ne), axis=1)
    onehot2 = col == i2[:, None]

    p = jnp.exp(logits - m1)
    z = jnp.sum(p, axis=1, keepdims=True)
    g1 = 1.0 / z
    g2 = jnp.exp(m2 - m1) / z
    den = jnp.maximum(g1 + g2, jnp.finfo(jnp.float32).eps)
    g1_ref[...] = (g1 / den)[:, 0]
    g2_ref[...] = (g2 / den)[:, 0]
    i1_ref[...] = i1
    i2_ref[...] = i2

    m1f = onehot1.astype(jnp.float32)
    m2f = onehot2.astype(jnp.float32)
    r = lax.broadcasted_iota(jnp.int32, (bt, bt), 0)
    c = lax.broadcasted_iota(jnp.int32, (bt, bt), 1)
    tri = (c <= r).astype(jnp.float32)
    cum1 = lax.dot_general(tri, m1f, (((1,), (0,)), ((), ())),
                           preferred_element_type=jnp.float32)
    cum2 = lax.dot_general(tri, m2f, (((1,), (0,)), ((), ())),
                           preferred_element_type=jnp.float32)
    c1 = cnt1_ref[...]
    c2 = cnt2_ref[...]
    l1_ref[...] = jnp.sum((cum1 - 1.0 + c1) * m1f, axis=1).astype(jnp.int32)
    l2_ref[...] = jnp.sum((cum2 - 1.0 + c2) * m2f, axis=1).astype(jnp.int32)
    new_c1 = c1 + cum1[bt - 1:bt, :]
    new_c2 = c2 + cum2[bt - 1:bt, :]
    new_me = me_ref[...] + jnp.sum(p / z, axis=0, keepdims=True)
    cnt1_ref[...] = new_c1
    cnt2_ref[...] = new_c2
    me_ref[...] = new_me

    @pl.when(step == nb - 1)
    def _fin():
        ntok = nb * bt
        loss_ref[...] = (jnp.sum(new_me * new_c1) * (ne / (ntok * ntok))
                         ).reshape(1, 1)
        ce_ref[...] = new_c1.astype(jnp.int32)


def _gate_call(x, wg, bt):
    nt, d = x.shape
    ne = wg.shape[0]
    nb = nt // bt
    tok = pl.BlockSpec((bt,), lambda i: (i,))
    return pl.pallas_call(
        _gate_body,
        grid=(nb,),
        in_specs=[pl.BlockSpec((bt, d), lambda i: (i, 0)),
                  pl.BlockSpec((ne, d), lambda i: (0, 0))],
        out_specs=[tok, tok, tok, tok, tok, tok,
                   pl.BlockSpec((1, 1), lambda i: (0, 0)),
                   pl.BlockSpec((1, ne), lambda i: (0, 0))],
        out_shape=[jax.ShapeDtypeStruct((nt,), jnp.float32),
                   jax.ShapeDtypeStruct((nt,), jnp.float32),
                   jax.ShapeDtypeStruct((nt,), jnp.int32),
                   jax.ShapeDtypeStruct((nt,), jnp.int32),
                   jax.ShapeDtypeStruct((nt,), jnp.int32),
                   jax.ShapeDtypeStruct((nt,), jnp.int32),
                   jax.ShapeDtypeStruct((1, 1), jnp.float32),
                   jax.ShapeDtypeStruct((1, ne), jnp.int32)],
        scratch_shapes=[pltpu.VMEM((1, ne), jnp.float32),
                        pltpu.VMEM((1, ne), jnp.float32),
                        pltpu.VMEM((1, ne), jnp.float32)],
    )(x, wg)


def _make_fixup(nt, ne, nw):
    chunk = nt // nw
    mesh = plsc.VectorSubcoreMesh(core_axis_name="c", subcore_axis_name="s")

    @functools.partial(
        pl.kernel, mesh=mesh,
        out_type=jax.ShapeDtypeStruct((nt,), jnp.int32),
        scratch_types=[pltpu.VMEM((chunk,), jnp.int32),
                       pltpu.VMEM((chunk,), jnp.int32),
                       pltpu.VMEM((chunk,), jnp.int32),
                       pltpu.SemaphoreType.DMA])
    def fixup(loc_hbm, idx_hbm, ce_hbm, out_hbm, loc_v, idx_v, gat_v, sem):
        wid = lax.axis_index("s") * 2 + lax.axis_index("c")
        base = wid * chunk
        pltpu.sync_copy(idx_hbm.at[pl.ds(base, chunk)], idx_v)
        pltpu.sync_copy(loc_hbm.at[pl.ds(base, chunk)], loc_v)
        pltpu.async_copy(ce_hbm.at[idx_v], gat_v, sem).wait()
        for i in range(chunk // 16):
            sl = pl.ds(i * 16, 16)
            loc_v[sl] = loc_v[sl] + gat_v[sl]
        pltpu.sync_copy(loc_v, out_hbm.at[pl.ds(base, chunk)])

    return fixup


def kernel(x, wg):
    nt, _ = x.shape
    ne = wg.shape[0]
    g1, g2, i1, i2, loc1, loc2p, loss, ce = _gate_call(x, wg, bt=512)
    ce1 = ce.reshape(ne)
    loc2 = _make_fixup(nt, ne, 32)(loc2p, i2, ce1)
    return loss.reshape(()), g1, g2, i1, i2, loc1, loc2


# speedup vs baseline: 1.5582x; 1.5582x over previous
import jax
import jax.numpy as jnp
from jax import lax
from jax.experimental import pallas as pl
from jax.experimental.pallas import tpu as pltpu


def _gate_body(x_ref, wg_ref, g1_ref, g2_ref, i1_ref, i2_ref, l1_ref, l2_ref,
               loss_ref, cnt1_ref, cnt2_ref, me_ref, i2s_ref, l2p_ref):
    step = pl.program_id(0)
    nb = pl.num_programs(0) - 1
    bt = x_ref.shape[0]
    ne = wg_ref.shape[0]

    @pl.when(step == 0)
    def _init():
        cnt1_ref[...] = jnp.zeros_like(cnt1_ref)
        cnt2_ref[...] = jnp.zeros_like(cnt2_ref)
        me_ref[...] = jnp.zeros_like(me_ref)

    @pl.when(step < nb)
    def _main():
        logits = lax.dot_general(
            x_ref[...].astype(jnp.bfloat16), wg_ref[...].astype(jnp.bfloat16),
            (((1,), (1,)), ((), ())), preferred_element_type=jnp.float32)

        col = lax.broadcasted_iota(jnp.int32, (bt, ne), 1)
        m1 = jnp.max(logits, axis=1, keepdims=True)
        i1 = jnp.min(jnp.where(logits == m1, col, ne), axis=1)
        onehot1 = col == i1[:, None]
        masked = jnp.where(onehot1, -jnp.inf, logits)
        m2 = jnp.max(masked, axis=1, keepdims=True)
        i2 = jnp.min(jnp.where(masked == m2, col, ne), axis=1)
        onehot2 = col == i2[:, None]

        p = jnp.exp(logits - m1)
        z = jnp.sum(p, axis=1, keepdims=True)
        g1 = 1.0 / z
        g2 = jnp.exp(m2 - m1) / z
        den = jnp.maximum(g1 + g2, jnp.finfo(jnp.float32).eps)
        g1_ref[...] = (g1 / den)[:, 0]
        g2_ref[...] = (g2 / den)[:, 0]
        i1_ref[...] = i1
        i2_ref[...] = i2

        m1f = onehot1.astype(jnp.float32)
        m2f = onehot2.astype(jnp.float32)
        r = lax.broadcasted_iota(jnp.int32, (bt, bt), 0)
        c = lax.broadcasted_iota(jnp.int32, (bt, bt), 1)
        tri = (c <= r).astype(jnp.float32)
        cum1 = lax.dot_general(tri, m1f, (((1,), (0,)), ((), ())),
                               preferred_element_type=jnp.float32)
        cum2 = lax.dot_general(tri, m2f, (((1,), (0,)), ((), ())),
                               preferred_element_type=jnp.float32)
        c1 = cnt1_ref[...]
        c2 = cnt2_ref[...]
        l1_ref[...] = jnp.sum((cum1 - 1.0 + c1) * m1f, axis=1).astype(jnp.int32)
        loc2p = jnp.sum((cum2 - 1.0 + c2) * m2f, axis=1).astype(jnp.int32)
        i2s_ref[pl.ds(step, 1), :] = i2[None, :]
        l2p_ref[pl.ds(step, 1), :] = loc2p[None, :]
        cnt1_ref[...] = c1 + cum1[bt - 1:bt, :]
        cnt2_ref[...] = c2 + cum2[bt - 1:bt, :]
        me_ref[...] = me_ref[...] + jnp.sum(p / z, axis=0, keepdims=True)

    @pl.when(step == nb)
    def _fin():
        ntok = nb * bt
        ce = cnt1_ref[...]
        loss_ref[...] = (jnp.sum(me_ref[...] * ce) * (ne / (ntok * ntok))
                         ).reshape(1, 1)
        col = lax.broadcasted_iota(jnp.int32, (bt, ne), 1)
        for cblk in range(nb):
            i2c = i2s_ref[cblk, :]
            oh = (col == i2c[:, None]).astype(jnp.float32)
            add = lax.dot_general(oh, ce, (((1,), (1,)), ((), ())),
                                  preferred_element_type=jnp.float32)
            l2_ref[pl.ds(cblk * bt, bt)] = (
                l2p_ref[cblk, :] + add[:, 0].astype(jnp.int32))


def _gate_call(x, wg, bt):
    nt, d = x.shape
    ne = wg.shape[0]
    nb = nt // bt
    tok = pl.BlockSpec((bt,), lambda i: (jnp.minimum(i, nb - 1),))
    return pl.pallas_call(
        _gate_body,
        grid=(nb + 1,),
        in_specs=[pl.BlockSpec((bt, d), lambda i: (jnp.minimum(i, nb - 1), 0)),
                  pl.BlockSpec((ne, d), lambda i: (0, 0))],
        out_specs=[tok, tok, tok, tok, tok,
                   pl.BlockSpec((nt,), lambda i: (0,)),
                   pl.BlockSpec((1, 1), lambda i: (0, 0))],
        out_shape=[jax.ShapeDtypeStruct((nt,), jnp.float32),
                   jax.ShapeDtypeStruct((nt,), jnp.float32),
                   jax.ShapeDtypeStruct((nt,), jnp.int32),
                   jax.ShapeDtypeStruct((nt,), jnp.int32),
                   jax.ShapeDtypeStruct((nt,), jnp.int32),
                   jax.ShapeDtypeStruct((nt,), jnp.int32),
                   jax.ShapeDtypeStruct((1, 1), jnp.float32)],
        scratch_shapes=[pltpu.VMEM((1, ne), jnp.float32),
                        pltpu.VMEM((1, ne), jnp.float32),
                        pltpu.VMEM((1, ne), jnp.float32),
                        pltpu.VMEM((nb, bt), jnp.int32),
                        pltpu.VMEM((nb, bt), jnp.int32)],
    )(x, wg)


def kernel(x, wg):
    g1, g2, i1, i2, loc1, loc2, loss = _gate_call(x, wg, bt=512)
    return loss.reshape(()), g1, g2, i1, i2, loc1, loc2
